# Initial kernel scaffold; baseline (speedup 1.0000x reference)
import jax
import jax.numpy as jnp
from jax.experimental import pallas as pl


def kernel(x, edge_index, W, b):
    raise NotImplementedError("write your pallas kernel here")


# speedup vs baseline: 27.5586x; 27.5586x over previous
import functools

import jax
import jax.numpy as jnp
from jax import lax
from jax.experimental import pallas as pl
from jax.experimental.pallas import tpu as pltpu
from jax.experimental.pallas import tpu_sc as plsc

N_NODES = 10000
N_EDGES = 320000
CH = 128

NC = 2
NS = 16
NW = NC * NS
EPW = N_EDGES // NW
ECH = 80
NCHUNK = EPW // ECH
RPT = 624
RPT_LAST = N_NODES - RPT * (NS - 1)

_MESH = plsc.VectorSubcoreMesh(
    core_axis_name="c", subcore_axis_name="s", num_cores=NC, num_subcores=NS)


def _deg_body(dst_hbm, ones_hbm, zeros_hbm, deg_out, dst_v, ones_v, deg_sh):
  c = lax.axis_index("c")
  s = lax.axis_index("s")
  w = s * NC + c

  @pl.when(s == 0)
  def _():
    pltpu.sync_copy(zeros_hbm, deg_sh)

  pltpu.sync_copy(dst_hbm.at[w], dst_v)
  pltpu.sync_copy(ones_hbm, ones_v)
  plsc.subcore_barrier()

  @pl.loop(0, NCHUNK)
  def _(j):
    pltpu.sync_copy(ones_v, deg_sh.at[dst_v.at[j]], add=True)

  plsc.subcore_barrier()

  @pl.when(s == 0)
  def _():
    pltpu.sync_copy(deg_sh, deg_out.at[c])


_deg_kernel = functools.partial(
    pl.kernel,
    out_type=jax.ShapeDtypeStruct((NC, N_NODES), jnp.float32),
    mesh=_MESH,
    scratch_types=[
        pltpu.VMEM((NCHUNK, ECH), jnp.int32),
        pltpu.VMEM((ECH,), jnp.float32),
        pltpu.VMEM_SHARED((N_NODES,), jnp.float32),
    ],
)(_deg_body)


def _edge_body(g_hbm, src_hbm, dst_hbm, zrow_hbm, out_hbm,
               src_v, dst_v, buf, acc):
  c = lax.axis_index("c")
  s = lax.axis_index("s")
  w = s * NC + c

  @pl.when(s < NS - 1)
  def _():
    pltpu.sync_copy(zrow_hbm.at[pl.ds(0, RPT)], acc.at[pl.ds(s * RPT, RPT)])

  @pl.when(s == NS - 1)
  def _():
    pltpu.sync_copy(zrow_hbm, acc.at[pl.ds((NS - 1) * RPT, RPT_LAST)])

  pltpu.sync_copy(src_hbm.at[w], src_v)
  pltpu.sync_copy(dst_hbm.at[w], dst_v)
  plsc.subcore_barrier()

  @pl.loop(0, NCHUNK)
  def _(j):
    pltpu.sync_copy(g_hbm.at[src_v.at[j]], buf)
    pltpu.sync_copy(buf, acc.at[dst_v.at[j]], add=True)

  plsc.subcore_barrier()

  @pl.when(s < NS - 1)
  def _():
    pltpu.sync_copy(acc.at[pl.ds(s * RPT, RPT)],
                    out_hbm.at[c, pl.ds(s * RPT, RPT)])

  @pl.when(s == NS - 1)
  def _():
    pltpu.sync_copy(acc.at[pl.ds((NS - 1) * RPT, RPT_LAST)],
                    out_hbm.at[c, pl.ds((NS - 1) * RPT, RPT_LAST)])


_edge_kernel = functools.partial(
    pl.kernel,
    out_type=jax.ShapeDtypeStruct((NC, N_NODES, CH), jnp.float32),
    mesh=_MESH,
    scratch_types=[
        pltpu.VMEM((NCHUNK, ECH), jnp.int32),
        pltpu.VMEM((NCHUNK, ECH), jnp.int32),
        pltpu.VMEM((ECH, CH), jnp.float32),
        pltpu.VMEM_SHARED((N_NODES, CH), jnp.float32),
    ],
)(_edge_body)


BM = 1000

def _dinv_block(deg_ref):
  dl = deg_ref[0]
  return lax.rsqrt(dl[0] + dl[1] + 1.0)


def _lin_body(deg_ref, x_ref, w_ref, g_ref):
  dinv = _dinv_block(deg_ref)
  h = jnp.dot(x_ref[...], w_ref[...], preferred_element_type=jnp.float32)
  g_ref[...] = h * dinv[:, None]


def _lin(deg2, x, W):
  return pl.pallas_call(
      _lin_body,
      grid=(N_NODES // BM,),
      in_specs=[
          pl.BlockSpec((1, NC, BM), lambda i: (i, 0, 0)),
          pl.BlockSpec((BM, CH), lambda i: (i, 0)),
          pl.BlockSpec((CH, CH), lambda i: (0, 0)),
      ],
      out_specs=pl.BlockSpec((BM, CH), lambda i: (i, 0)),
      out_shape=jax.ShapeDtypeStruct((N_NODES, CH), jnp.float32),
  )(deg2, x, W)


def _fin_body(deg_ref, p_ref, g_ref, b_ref, o_ref):
  dinv = _dinv_block(deg_ref)
  t = (p_ref[0] + p_ref[1] + g_ref[...]) * dinv[:, None] + b_ref[...]
  o_ref[...] = jnp.maximum(t, 0.0)


def _fin(deg2, P, g, b2):
  return pl.pallas_call(
      _fin_body,
      grid=(N_NODES // BM,),
      in_specs=[
          pl.BlockSpec((1, NC, BM), lambda i: (i, 0, 0)),
          pl.BlockSpec((NC, BM, CH), lambda i: (0, i, 0)),
          pl.BlockSpec((BM, CH), lambda i: (i, 0)),
          pl.BlockSpec((1, CH), lambda i: (0, 0)),
      ],
      out_specs=pl.BlockSpec((BM, CH), lambda i: (i, 0)),
      out_shape=jax.ShapeDtypeStruct((N_NODES, CH), jnp.float32),
  )(deg2, P, g, b2)


@jax.jit
def kernel(x, edge_index, W, b):
  src = edge_index[0].astype(jnp.int32).reshape(NW, NCHUNK, ECH)
  dst = edge_index[1].astype(jnp.int32).reshape(NW, NCHUNK, ECH)
  ones_c = jnp.ones((ECH,), jnp.float32)
  zeros_n = jnp.zeros((N_NODES,), jnp.float32)
  zrow = jnp.zeros((RPT_LAST, CH), jnp.float32)

  deg2 = _deg_kernel(dst, ones_c, zeros_n)
  deg2 = deg2.reshape(NC, N_NODES // BM, BM).transpose(1, 0, 2)
  g = _lin(deg2, x, W)
  P = _edge_kernel(g, src, dst, zrow)
  return _fin(deg2, P, g, b.reshape(1, CH))
